# baseline (speedup 1.0000x reference)
import functools

import jax
import jax.numpy as jnp
from jax import lax
from jax.experimental import pallas as pl
from jax.experimental.pallas import tpu as pltpu
from jax.experimental.pallas import tpu_sc as plsc

NC = 2
NS = 16
LANES = 16
NW = NC * NS
BATCH = 16384
BPW = BATCH // NW
DIM = 32
HALF = DIM // 2
WIDE = 128
PACK = WIDE // DIM
CHUNK = 128
NCHUNK = BPW // CHUNK
NBUF = 2


def _sc_body(head_h, rel_h, tail_h, ent_h, remb_h, out_h,
             hoidx, toidx, roidx, hwidx, twidx, rwidx, bufs, scores, sems):
    wid = lax.axis_index("s") * NC + lax.axis_index("c")
    base = wid * BPW

    oidx = (hoidx, toidx, roidx)
    widx = (hwidx, twidx, rwidx)
    for t, idx_h in enumerate((head_h, tail_h, rel_h)):
        pltpu.sync_copy(idx_h.at[pl.ds(base, BPW)], oidx[t])
        for v in range(BPW // LANES):
            sl = pl.ds(v * LANES, LANES)
            widx[t][sl] = jax.lax.shift_right_logical(oidx[t][sl], 2)

    tables = (ent_h, ent_h, remb_h)

    def fire(j, slot):
        for t in range(3):
            pltpu.async_copy(
                tables[t].at[widx[t].at[pl.ds(j * CHUNK, CHUNK)]],
                bufs.at[slot, t],
                sems.at[slot, t],
            )

    def wait(slot):
        for t in range(3):
            pltpu.make_async_copy(
                tables[t].at[widx[t].at[pl.ds(0, CHUNK)]],
                bufs.at[slot, t],
                sems.at[slot, t],
            ).wait()

    lane_iota = lax.iota(jnp.int32, LANES)
    zero16 = jnp.zeros((LANES,), jnp.float32)

    def compute(j, slot):
        for b in range(CHUNK // LANES):
            e_vec = b * LANES + lane_iota
            sl = pl.ds(j * CHUNK + b * LANES, LANES)
            hsub = (hoidx[sl] & 3) * DIM
            tsub = (toidx[sl] & 3) * DIM
            rsub = (roidx[sl] & 3) * DIM
            acc = zero16
            for d in range(HALF):
                hre = plsc.load_gather(bufs.at[slot, 0], [e_vec, hsub + d])
                him = plsc.load_gather(bufs.at[slot, 0], [e_vec, hsub + d + HALF])
                tre = plsc.load_gather(bufs.at[slot, 1], [e_vec, tsub + d])
                tim = plsc.load_gather(bufs.at[slot, 1], [e_vec, tsub + d + HALF])
                rre = plsc.load_gather(bufs.at[slot, 2], [e_vec, rsub + d])
                rim = plsc.load_gather(bufs.at[slot, 2], [e_vec, rsub + d + HALF])
                real = hre * rre - him * rim
                imag = hre * rim + him * rre
                acc = acc + (tre * real - tim * imag)
            scores[sl] = 1.0 / (1.0 + jnp.exp(-acc))

    fire(0, 0)
    for j in range(NCHUNK):
        if j + 1 < NCHUNK:
            fire(j + 1, (j + 1) % NBUF)
        wait(j % NBUF)
        compute(j, j % NBUF)

    pltpu.sync_copy(scores, out_h.at[pl.ds(base, BPW)])


@functools.cache
def _build_sc_kernel():
    return pl.kernel(
        _sc_body,
        out_type=jax.ShapeDtypeStruct((BATCH,), jnp.float32),
        mesh=plsc.VectorSubcoreMesh(
            core_axis_name="c", subcore_axis_name="s",
            num_cores=NC, num_subcores=NS),
        compiler_params=pltpu.CompilerParams(
            needs_layout_passes=False, use_tc_tiling_on_sc=True),
        scratch_types=[
            pltpu.VMEM((BPW,), jnp.int32),
            pltpu.VMEM((BPW,), jnp.int32),
            pltpu.VMEM((BPW,), jnp.int32),
            pltpu.VMEM((BPW,), jnp.int32),
            pltpu.VMEM((BPW,), jnp.int32),
            pltpu.VMEM((BPW,), jnp.int32),
            pltpu.VMEM((NBUF, 3, CHUNK, WIDE), jnp.float32),
            pltpu.VMEM((BPW,), jnp.float32),
            pltpu.SemaphoreType.DMA((NBUF, 3)),
        ],
    )


def kernel(head, relation, tail, entity_embed, relation_embed):
    ent4 = entity_embed.reshape(entity_embed.shape[0] // PACK, WIDE)
    rel4 = relation_embed.reshape(relation_embed.shape[0] // PACK, WIDE)
    return _build_sc_kernel()(head, relation, tail, ent4, rel4)


# speedup vs baseline: 1.4091x; 1.4091x over previous
import functools

import jax
import jax.numpy as jnp
from jax import lax
from jax.experimental import pallas as pl
from jax.experimental.pallas import tpu as pltpu
from jax.experimental.pallas import tpu_sc as plsc

NC = 2
NS = 16
LANES = 16
NW = NC * NS
BATCH = 16384
BPW = BATCH // NW
DIM = 32
HALF = DIM // 2
SLAB = 8
WIDE = 128
PACK = WIDE // DIM
RCHUNK = 128
NWAVE = BPW // LANES
NBUF = 2


def _sc_body(head_h, rel_h, tail_h, ent_h, rel4_h, out_h,
             hoidx, toidx, roidx, rwidx,
             hslab, tslab, relchunk, rcomp, scores, sems, rsem):
    wid = lax.axis_index("s") * NC + lax.axis_index("c")
    base = wid * BPW

    pltpu.sync_copy(head_h.at[pl.ds(base, BPW)], hoidx)
    pltpu.sync_copy(tail_h.at[pl.ds(base, BPW)], toidx)
    pltpu.sync_copy(rel_h.at[pl.ds(base, BPW)], roidx)


    lane_iota = lax.iota(jnp.int32, LANES)
    for v in range(BPW // LANES):
        sl = pl.ds(v * LANES, LANES)
        rwidx[sl] = jax.lax.shift_right_logical(roidx[sl], 2)
    for j in range(BPW // RCHUNK):
        pltpu.async_copy(
            rel4_h.at[rwidx.at[pl.ds(j * RCHUNK, RCHUNK)]],
            relchunk, rsem,
        ).wait()
        for b in range(RCHUNK // LANES):
            sl = pl.ds(j * RCHUNK + b * LANES, LANES)
            rsub = (roidx[sl] & (PACK - 1)) * DIM
            ev = b * LANES + lane_iota
            for d in range(DIM):
                rcomp[d, sl] = plsc.load_gather(relchunk, [ev, rsub + d])

    def fire(w, slot):
        hv = hoidx[pl.ds(w * LANES, LANES)]
        tv = toidx[pl.ds(w * LANES, LANES)]
        hv = jax.lax.shift_right_logical(hv, 3) * SLAB
        tv = jax.lax.shift_right_logical(tv, 3) * SLAB
        for j in range(LANES):
            hi = pl.multiple_of(hv[j], SLAB)
            ti = pl.multiple_of(tv[j], SLAB)
            pltpu.async_copy(
                ent_h.at[pl.ds(hi, SLAB), :],
                hslab.at[slot, j], sems.at[slot, 0])
            pltpu.async_copy(
                ent_h.at[pl.ds(ti, SLAB), :],
                tslab.at[slot, j], sems.at[slot, 1])

    def wait(slot):
        for j in range(LANES):
            pltpu.make_async_copy(
                ent_h.at[pl.ds(0, SLAB), :], hslab.at[slot, j],
                sems.at[slot, 0]).wait()
            pltpu.make_async_copy(
                ent_h.at[pl.ds(0, SLAB), :], tslab.at[slot, j],
                sems.at[slot, 1]).wait()

    zero16 = jnp.zeros((LANES,), jnp.float32)

    def compute(w, slot):
        sl = pl.ds(w * LANES, LANES)
        hrow = hoidx[sl] & (SLAB - 1)
        trow = toidx[sl] & (SLAB - 1)
        acc = zero16
        for d in range(HALF):
            cold = jnp.full((LANES,), d, jnp.int32)
            coldi = jnp.full((LANES,), d + HALF, jnp.int32)
            hre = plsc.load_gather(hslab.at[slot], [lane_iota, hrow, cold])
            him = plsc.load_gather(hslab.at[slot], [lane_iota, hrow, coldi])
            tre = plsc.load_gather(tslab.at[slot], [lane_iota, trow, cold])
            tim = plsc.load_gather(tslab.at[slot], [lane_iota, trow, coldi])
            rre = rcomp[d, sl]
            rim = rcomp[d + HALF, sl]
            real = hre * rre - him * rim
            imag = hre * rim + him * rre
            acc = acc + (tre * real - tim * imag)
        scores[sl] = 1.0 / (1.0 + jnp.exp(-acc))

    fire(0, 0)

    def wave(w, carry):
        slot = w % NBUF

        @pl.when(w + 1 < NWAVE)
        def _():
            fire(w + 1, (w + 1) % NBUF)

        wait(slot)
        compute(w, slot)
        return carry

    lax.fori_loop(0, NWAVE, wave, 0)

    pltpu.sync_copy(scores, out_h.at[pl.ds(base, BPW)])


@functools.cache
def _build_sc_kernel():
    return pl.kernel(
        _sc_body,
        out_type=jax.ShapeDtypeStruct((BATCH,), jnp.float32),
        mesh=plsc.VectorSubcoreMesh(
            core_axis_name="c", subcore_axis_name="s",
            num_cores=NC, num_subcores=NS),
        compiler_params=pltpu.CompilerParams(
            needs_layout_passes=False, use_tc_tiling_on_sc=True),
        scratch_types=[
            pltpu.VMEM((BPW,), jnp.int32),
            pltpu.VMEM((BPW,), jnp.int32),
            pltpu.VMEM((BPW,), jnp.int32),
            pltpu.VMEM((BPW,), jnp.int32),
            pltpu.VMEM((NBUF, LANES, SLAB, DIM), jnp.float32),
            pltpu.VMEM((NBUF, LANES, SLAB, DIM), jnp.float32),
            pltpu.VMEM((RCHUNK, WIDE), jnp.float32),
            pltpu.VMEM((DIM, BPW), jnp.float32),
            pltpu.VMEM((BPW,), jnp.float32),
            pltpu.SemaphoreType.DMA((NBUF, 2)),
            pltpu.SemaphoreType.DMA,
        ],
    )


def kernel(head, relation, tail, entity_embed, relation_embed):
    rel4 = relation_embed.reshape(relation_embed.shape[0] // PACK, WIDE)
    return _build_sc_kernel()(head, relation, tail, entity_embed, rel4)
